# baseline (speedup 1.0000x reference)
import functools

import jax
import jax.numpy as jnp
from jax import lax
from jax.experimental import pallas as pl
from jax.experimental.pallas import tpu as pltpu
from jax.experimental.pallas import tpu_sc as plsc

N = 16384
NUM_CLASSES = 1000
NC = 1
NS = 16
NW = NC * NS
RPW = N // NW
R = 32
G = RPW // R
NRING = 3

_mesh = plsc.VectorSubcoreMesh(core_axis_name="c", subcore_axis_name="s", num_cores=NC)


@functools.partial(
    pl.kernel,
    out_type=jax.ShapeDtypeStruct((N, NUM_CLASSES), jnp.int32),
    mesh=_mesh,
    compiler_params=pltpu.CompilerParams(
        use_tc_tiling_on_sc=True, needs_layout_passes=False
    ),
    scratch_types=[
        pltpu.VMEM((RPW,), jnp.int32),
        pltpu.VMEM((R, NUM_CLASSES), jnp.int32),
        pltpu.VMEM((R, NUM_CLASSES), jnp.int32),
        pltpu.VMEM((R, NUM_CLASSES), jnp.int32),
        pltpu.SemaphoreType.DMA,
        pltpu.SemaphoreType.DMA,
        pltpu.SemaphoreType.DMA,
    ],
)
def _sc_onehot(x_hbm, out_hbm, idx_v, buf0, buf1, buf2, sem0, sem1, sem2):
    bufs = (buf0, buf1, buf2)
    sems = (sem0, sem1, sem2)
    wid = lax.axis_index("s") * NC + lax.axis_index("c")
    base = wid * RPW

    pltpu.sync_copy(x_hbm.at[pl.ds(base, RPW)], idx_v)

    zeros = jnp.zeros((16,), jnp.int32)
    ones = jnp.full((16,), 1, jnp.int32)
    lane = lax.broadcasted_iota(jnp.int32, (16,), 0)

    def _zero_row(r, carry):
        for buf in bufs:
            for c in range(62):
                buf[r, pl.ds(c * 16, 16)] = zeros
            buf[r, pl.ds(NUM_CLASSES - 16, 16)] = zeros
        return carry

    lax.fori_loop(0, R, _zero_row, 0)

    def _scatter(buf, g, val):
        for t in range(R // 16):
            cols = idx_v[pl.ds(g * R + t * 16, 16)]
            plsc.store_scatter(buf, [t * 16 + lane, cols], val)

    for g in range(G):
        b = g % NRING
        if g >= NRING:
            pltpu.make_async_copy(
                bufs[b],
                out_hbm.at[pl.ds(base + (g - NRING) * R, R), :],
                sems[b],
            ).wait()
            _scatter(bufs[b], g - NRING, zeros)
        _scatter(bufs[b], g, ones)
        pltpu.make_async_copy(
            bufs[b],
            out_hbm.at[pl.ds(base + g * R, R), :],
            sems[b],
        ).start()

    for g in range(G - NRING, G):
        b = g % NRING
        pltpu.make_async_copy(
            bufs[b],
            out_hbm.at[pl.ds(base + g * R, R), :],
            sems[b],
        ).wait()


def kernel(x):
    return _sc_onehot(x)


# speedup vs baseline: 1.1795x; 1.1795x over previous
import functools

import jax
import jax.numpy as jnp
from jax import lax
from jax.experimental import pallas as pl
from jax.experimental.pallas import tpu as pltpu
from jax.experimental.pallas import tpu_sc as plsc

N = 16384
NUM_CLASSES = 1000
NC = 2
NS = 16
NW = NC * NS
RPW = N // NW
R = 32
G = RPW // R
NRING = 3

_mesh = plsc.VectorSubcoreMesh(core_axis_name="c", subcore_axis_name="s", num_cores=NC)


@functools.partial(
    pl.kernel,
    out_type=jax.ShapeDtypeStruct((N, NUM_CLASSES), jnp.int32),
    mesh=_mesh,
    compiler_params=pltpu.CompilerParams(
        use_tc_tiling_on_sc=True, needs_layout_passes=False
    ),
    scratch_types=[
        pltpu.VMEM((RPW,), jnp.int32),
        pltpu.VMEM((R, NUM_CLASSES), jnp.int32),
        pltpu.VMEM((R, NUM_CLASSES), jnp.int32),
        pltpu.VMEM((R, NUM_CLASSES), jnp.int32),
        pltpu.SemaphoreType.DMA,
        pltpu.SemaphoreType.DMA,
        pltpu.SemaphoreType.DMA,
    ],
)
def _sc_onehot(x_hbm, out_hbm, idx_v, buf0, buf1, buf2, sem0, sem1, sem2):
    bufs = (buf0, buf1, buf2)
    sems = (sem0, sem1, sem2)
    wid = lax.axis_index("s") * NC + lax.axis_index("c")
    base = wid * RPW

    pltpu.sync_copy(x_hbm.at[pl.ds(base, RPW)], idx_v)

    zeros = jnp.zeros((16,), jnp.int32)
    ones = jnp.full((16,), 1, jnp.int32)
    lane = lax.broadcasted_iota(jnp.int32, (16,), 0)

    def _zero_row(r, carry):
        for buf in bufs:
            for c in range(62):
                buf[r, pl.ds(c * 16, 16)] = zeros
            buf[r, pl.ds(NUM_CLASSES - 16, 16)] = zeros
        return carry

    lax.fori_loop(0, R, _zero_row, 0)

    def _scatter(buf, g, val):
        for t in range(R // 16):
            cols = idx_v[pl.ds(g * R + t * 16, 16)]
            plsc.store_scatter(buf, [t * 16 + lane, cols], val)

    for g in range(G):
        b = g % NRING
        if g >= NRING:
            pltpu.make_async_copy(
                bufs[b],
                out_hbm.at[pl.ds(base + (g - NRING) * R, R), :],
                sems[b],
            ).wait()
            _scatter(bufs[b], g - NRING, zeros)
        _scatter(bufs[b], g, ones)
        pltpu.make_async_copy(
            bufs[b],
            out_hbm.at[pl.ds(base + g * R, R), :],
            sems[b],
        ).start()

    for g in range(G - NRING, G):
        b = g % NRING
        pltpu.make_async_copy(
            bufs[b],
            out_hbm.at[pl.ds(base + g * R, R), :],
            sems[b],
        ).wait()


def kernel(x):
    return _sc_onehot(x)
